# Initial kernel scaffold; baseline (speedup 1.0000x reference)
import jax
import jax.numpy as jnp
from jax.experimental import pallas as pl


def kernel(feature, nodes, n1, nn_seed, nn_n1, W0, b0, W1, b1):
    raise NotImplementedError("write your pallas kernel here")


# speedup vs baseline: 7.4047x; 7.4047x over previous
import functools

import jax
import jax.numpy as jnp
from jax import lax
from jax.experimental import pallas as pl
from jax.experimental.pallas import tpu as pltpu
from jax.experimental.pallas import tpu_sc as plsc

_N, _D = 100000, 128
_B, _S1, _S2 = 1024, 25, 10

_NC, _NS = 2, 16
_NW = _NC * _NS

_PAIRS = _B * _S2 // _NW
_SEEDS = _B // _NW
_CH = 8
_ROWS = _CH * _S1
_NCHUNK_N1 = _PAIRS // _CH
_NCHUNK_SEED = _SEEDS // _CH
_VREGS = _D // 16


def _sc_gather_body(feat, nnn1, nnseed, n1f, nodes,
                    sum_n1, sum_seed, self_n1, self_seed,
                    idx_v, buf0, buf1, acc_v, acc2_v, sem0, sem1, osem):
  wid = lax.axis_index("c") * _NS + lax.axis_index("s")

  def gather_pieces(rows):
    pieces, o = [], 0
    while o < rows:
      ln = min(128, rows - o)
      pieces.append((o, ln))
      o += ln
    return pieces

  def gather_start(idx_off, buf, sem, rows):
    for o, ln in gather_pieces(rows):
      pltpu.async_copy(feat.at[idx_v.at[pl.ds(idx_off + o, ln)]],
                       buf.at[pl.ds(o, ln)], sem)

  def gather_wait(idx_off, buf, sem, rows):
    for o, ln in gather_pieces(rows):
      pltpu.make_async_copy(feat.at[idx_v.at[pl.ds(idx_off + o, ln)]],
                            buf.at[pl.ds(o, ln)], sem).wait()

  def acc_group(buf, acc_ref, out_base):
    def pair_body(p, carry):
      for v in range(_VREGS):
        s = buf[p * _S1, pl.ds(v * 16, 16)]
        for j in range(1, _S1):
          s = s + buf[p * _S1 + j, pl.ds(v * 16, 16)]
        acc_ref[out_base + p, pl.ds(v * 16, 16)] = s
      return carry
    lax.fori_loop(0, _CH, pair_body, 0)

  def sum_phase(idx_hbm, idx_count, nchunks, acc_ref):
    base = wid * idx_count
    pltpu.sync_copy(idx_hbm.at[pl.ds(base, idx_count)],
                    idx_v.at[pl.ds(0, idx_count)])
    gather_start(0, buf0, sem0, _ROWS)
    gather_start(_ROWS, buf1, sem1, _ROWS)

    def loop_body(k, carry):
      g0 = 2 * k
      gather_wait(g0 * _ROWS, buf0, sem0, _ROWS)
      acc_group(buf0, acc_ref, g0 * _CH)

      @pl.when(g0 + 2 < nchunks)
      def _():
        gather_start((g0 + 2) * _ROWS, buf0, sem0, _ROWS)

      gather_wait((g0 + 1) * _ROWS, buf1, sem1, _ROWS)
      acc_group(buf1, acc_ref, (g0 + 1) * _CH)

      @pl.when(g0 + 3 < nchunks)
      def _():
        gather_start((g0 + 3) * _ROWS, buf1, sem1, _ROWS)
      return carry

    lax.fori_loop(0, nchunks // 2, loop_body, 0)

  sum_phase(nnn1, _PAIRS * _S1, _NCHUNK_N1, acc_v)
  pltpu.async_copy(acc_v, sum_n1.at[pl.ds(wid * _PAIRS, _PAIRS)], osem)

  sum_phase(nnseed, _SEEDS * _S1, _NCHUNK_SEED, acc2_v)
  pltpu.async_copy(acc2_v, sum_seed.at[pl.ds(wid * _SEEDS, _SEEDS)], osem)

  pltpu.make_async_copy(acc_v, sum_n1.at[pl.ds(wid * _PAIRS, _PAIRS)], osem).wait()
  pltpu.sync_copy(n1f.at[pl.ds(wid * _PAIRS, _PAIRS)], idx_v.at[pl.ds(0, _PAIRS)])
  gather_start(0, acc_v, sem0, _PAIRS)
  gather_wait(0, acc_v, sem0, _PAIRS)
  pltpu.async_copy(acc_v.at[pl.ds(0, _PAIRS)],
                   self_n1.at[pl.ds(wid * _PAIRS, _PAIRS)], osem)

  pltpu.make_async_copy(acc2_v, sum_seed.at[pl.ds(wid * _SEEDS, _SEEDS)], osem).wait()
  pltpu.sync_copy(nodes.at[pl.ds(wid * _SEEDS, _SEEDS)], idx_v.at[pl.ds(0, _SEEDS)])
  gather_start(0, acc2_v, sem1, _SEEDS)
  gather_wait(0, acc2_v, sem1, _SEEDS)
  pltpu.async_copy(acc2_v, self_seed.at[pl.ds(wid * _SEEDS, _SEEDS)], osem)

  pltpu.make_async_copy(acc_v.at[pl.ds(0, _PAIRS)],
                        self_n1.at[pl.ds(wid * _PAIRS, _PAIRS)], osem).wait()
  pltpu.make_async_copy(acc2_v, self_seed.at[pl.ds(wid * _SEEDS, _SEEDS)],
                        osem).wait()


def _sc_gather(feature, nnn1f, nnseedf, n1f, nodes):
  mesh = plsc.VectorSubcoreMesh(core_axis_name="c", subcore_axis_name="s",
                                num_cores=_NC, num_subcores=_NS)
  f32 = jnp.float32
  return pl.kernel(
      _sc_gather_body,
      out_type=(
          jax.ShapeDtypeStruct((_B * _S2, _D), f32),
          jax.ShapeDtypeStruct((_B, _D), f32),
          jax.ShapeDtypeStruct((_B * _S2, _D), f32),
          jax.ShapeDtypeStruct((_B, _D), f32),
      ),
      mesh=mesh,
      scratch_types=[
          pltpu.VMEM((_PAIRS * _S1,), jnp.int32),
          pltpu.VMEM((_ROWS, _D), f32),
          pltpu.VMEM((_ROWS, _D), f32),
          pltpu.VMEM((_PAIRS, _D), f32),
          pltpu.VMEM((_SEEDS, _D), f32),
          pltpu.SemaphoreType.DMA,
          pltpu.SemaphoreType.DMA,
          pltpu.SemaphoreType.DMA,
      ],
  )(feature, nnn1f, nnseedf, n1f, nodes)


_BS = 128


def _l2norm(h):
  ss = jnp.sum(h * h, axis=-1, keepdims=True)
  return h / jnp.maximum(jnp.sqrt(ss), 1e-12)


def _tc_body(ss_ref, sums_ref, sn_ref, sumn_ref, w0_ref, b0_ref, w1_ref,
             b1_ref, o_ref):
  f32 = jnp.float32
  w0 = w0_ref[:]
  w0a, w0b = w0[:_D], w0[_D:]
  b0 = b0_ref[:]
  inv_s1 = 1.0 / _S1

  hs = jnp.dot(ss_ref[:], w0a, preferred_element_type=f32)
  hs += jnp.dot(sums_ref[:] * inv_s1, w0b, preferred_element_type=f32)
  hs = _l2norm(jnp.maximum(hs + b0, 0.0))

  hn = jnp.dot(sn_ref[:], w0a, preferred_element_type=f32)
  hn += jnp.dot(sumn_ref[:] * inv_s1, w0b, preferred_element_type=f32)
  hn = _l2norm(jnp.maximum(hn + b0, 0.0))

  m = jnp.mean(hn.reshape(_BS, _S2, _D), axis=1)
  w1 = w1_ref[:]
  z = jnp.dot(hs, w1[:_D], preferred_element_type=f32)
  z += jnp.dot(m, w1[_D:], preferred_element_type=f32)
  o_ref[:] = _l2norm(jnp.maximum(z + b1_ref[:], 0.0))


def _tc_mlp(self_seed, sum_seed, self_n1, sum_n1, W0, b0, W1, b1):
  grid = (_B // _BS,)
  rep = lambda i: (0, 0)
  return pl.pallas_call(
      _tc_body,
      grid=grid,
      in_specs=[
          pl.BlockSpec((_BS, _D), lambda i: (i, 0)),
          pl.BlockSpec((_BS, _D), lambda i: (i, 0)),
          pl.BlockSpec((_BS * _S2, _D), lambda i: (i, 0)),
          pl.BlockSpec((_BS * _S2, _D), lambda i: (i, 0)),
          pl.BlockSpec((2 * _D, _D), rep),
          pl.BlockSpec((1, _D), rep),
          pl.BlockSpec((2 * _D, _D), rep),
          pl.BlockSpec((1, _D), rep),
      ],
      out_specs=pl.BlockSpec((_BS, _D), lambda i: (i, 0)),
      out_shape=jax.ShapeDtypeStruct((_B, _D), jnp.float32),
  )(self_seed, sum_seed, self_n1, sum_n1, W0, b0, W1, b1)


@jax.jit
def kernel(feature, nodes, n1, nn_seed, nn_n1, W0, b0, W1, b1):
  nodes = nodes.reshape(-1).astype(jnp.int32)
  n1f = n1.reshape(-1).astype(jnp.int32)
  nnseedf = nn_seed.reshape(-1).astype(jnp.int32)
  nnn1f = nn_n1.reshape(-1).astype(jnp.int32)
  sum_n1, sum_seed, self_n1, self_seed = _sc_gather(
      feature, nnn1f, nnseedf, n1f, nodes)
  return _tc_mlp(self_seed, sum_seed, self_n1, sum_n1,
                 W0, b0.reshape(1, _D), W1, b1.reshape(1, _D))


# speedup vs baseline: 9.5981x; 1.2962x over previous
import functools

import jax
import jax.numpy as jnp
from jax import lax
from jax.experimental import pallas as pl
from jax.experimental.pallas import tpu as pltpu
from jax.experimental.pallas import tpu_sc as plsc

_N, _D = 100000, 128
_B, _S1, _S2 = 1024, 25, 10

_NC, _NS = 2, 16
_NW = _NC * _NS

_PAIRS = _B * _S2 // _NW
_SEEDS = _B // _NW
_CH = 8
_ROWS = _CH * _S1
_NCHUNK_N1 = _PAIRS // _CH
_NCHUNK_SEED = _SEEDS // _CH
_VREGS = _D // 16


def _sc_gather_body(feat, nnn1, nnseed, n1f, nodes,
                    sum_n1, sum_seed, self_n1, self_seed,
                    idx_v, buf0, buf1, acc_v, acc2_v, sem0, sem1, osem):
  wid = lax.axis_index("c") * _NS + lax.axis_index("s")

  def gather_pieces(rows):
    pieces, o = [], 0
    while o < rows:
      ln = min(128, rows - o)
      pieces.append((o, ln))
      o += ln
    return pieces

  def gather_start(idx_off, buf, sem, rows):
    for o, ln in gather_pieces(rows):
      pltpu.async_copy(feat.at[idx_v.at[pl.ds(idx_off + o, ln)]],
                       buf.at[pl.ds(o, ln)], sem)

  def gather_wait(idx_off, buf, sem, rows):
    for o, ln in gather_pieces(rows):
      pltpu.make_async_copy(feat.at[idx_v.at[pl.ds(idx_off + o, ln)]],
                            buf.at[pl.ds(o, ln)], sem).wait()

  def acc_group(buf, acc_ref, out_base):
    def pair_body(p, carry):
      s = [buf[p * _S1, pl.ds(v * 16, 16)] for v in range(_VREGS)]
      for j in range(1, _S1):
        for v in range(_VREGS):
          s[v] = s[v] + buf[p * _S1 + j, pl.ds(v * 16, 16)]
      for v in range(_VREGS):
        acc_ref[out_base + p, pl.ds(v * 16, 16)] = s[v]
      return carry
    lax.fori_loop(0, _CH, pair_body, 0)

  def sum_phase(idx_hbm, idx_count, nchunks, acc_ref):
    base = wid * idx_count
    pltpu.sync_copy(idx_hbm.at[pl.ds(base, idx_count)],
                    idx_v.at[pl.ds(0, idx_count)])
    gather_start(0, buf0, sem0, _ROWS)
    gather_start(_ROWS, buf1, sem1, _ROWS)

    def loop_body(k, carry):
      g0 = 2 * k
      gather_wait(g0 * _ROWS, buf0, sem0, _ROWS)
      acc_group(buf0, acc_ref, g0 * _CH)

      @pl.when(g0 + 2 < nchunks)
      def _():
        gather_start((g0 + 2) * _ROWS, buf0, sem0, _ROWS)

      gather_wait((g0 + 1) * _ROWS, buf1, sem1, _ROWS)
      acc_group(buf1, acc_ref, (g0 + 1) * _CH)

      @pl.when(g0 + 3 < nchunks)
      def _():
        gather_start((g0 + 3) * _ROWS, buf1, sem1, _ROWS)
      return carry

    lax.fori_loop(0, nchunks // 2, loop_body, 0)

  sum_phase(nnn1, _PAIRS * _S1, _NCHUNK_N1, acc_v)
  pltpu.async_copy(acc_v, sum_n1.at[pl.ds(wid * _PAIRS, _PAIRS)], osem)

  sum_phase(nnseed, _SEEDS * _S1, _NCHUNK_SEED, acc2_v)
  pltpu.async_copy(acc2_v, sum_seed.at[pl.ds(wid * _SEEDS, _SEEDS)], osem)

  pltpu.make_async_copy(acc_v, sum_n1.at[pl.ds(wid * _PAIRS, _PAIRS)], osem).wait()
  pltpu.sync_copy(n1f.at[pl.ds(wid * _PAIRS, _PAIRS)], idx_v.at[pl.ds(0, _PAIRS)])
  gather_start(0, acc_v, sem0, _PAIRS)
  gather_wait(0, acc_v, sem0, _PAIRS)
  pltpu.async_copy(acc_v.at[pl.ds(0, _PAIRS)],
                   self_n1.at[pl.ds(wid * _PAIRS, _PAIRS)], osem)

  pltpu.make_async_copy(acc2_v, sum_seed.at[pl.ds(wid * _SEEDS, _SEEDS)], osem).wait()
  pltpu.sync_copy(nodes.at[pl.ds(wid * _SEEDS, _SEEDS)], idx_v.at[pl.ds(0, _SEEDS)])
  gather_start(0, acc2_v, sem1, _SEEDS)
  gather_wait(0, acc2_v, sem1, _SEEDS)
  pltpu.async_copy(acc2_v, self_seed.at[pl.ds(wid * _SEEDS, _SEEDS)], osem)

  pltpu.make_async_copy(acc_v.at[pl.ds(0, _PAIRS)],
                        self_n1.at[pl.ds(wid * _PAIRS, _PAIRS)], osem).wait()
  pltpu.make_async_copy(acc2_v, self_seed.at[pl.ds(wid * _SEEDS, _SEEDS)],
                        osem).wait()


def _sc_gather(feature, nnn1f, nnseedf, n1f, nodes):
  mesh = plsc.VectorSubcoreMesh(core_axis_name="c", subcore_axis_name="s",
                                num_cores=_NC, num_subcores=_NS)
  f32 = jnp.float32
  return pl.kernel(
      _sc_gather_body,
      out_type=(
          jax.ShapeDtypeStruct((_B * _S2, _D), f32),
          jax.ShapeDtypeStruct((_B, _D), f32),
          jax.ShapeDtypeStruct((_B * _S2, _D), f32),
          jax.ShapeDtypeStruct((_B, _D), f32),
      ),
      mesh=mesh,
      scratch_types=[
          pltpu.VMEM((_PAIRS * _S1,), jnp.int32),
          pltpu.VMEM((_ROWS, _D), f32),
          pltpu.VMEM((_ROWS, _D), f32),
          pltpu.VMEM((_PAIRS, _D), f32),
          pltpu.VMEM((_SEEDS, _D), f32),
          pltpu.SemaphoreType.DMA,
          pltpu.SemaphoreType.DMA,
          pltpu.SemaphoreType.DMA,
      ],
  )(feature, nnn1f, nnseedf, n1f, nodes)


_BS = 128


def _l2norm(h):
  ss = jnp.sum(h * h, axis=-1, keepdims=True)
  return h / jnp.maximum(jnp.sqrt(ss), 1e-12)


def _tc_body(ss_ref, sums_ref, sn_ref, sumn_ref, w0_ref, b0_ref, w1_ref,
             b1_ref, o_ref):
  f32 = jnp.float32
  w0 = w0_ref[:]
  w0a, w0b = w0[:_D], w0[_D:]
  b0 = b0_ref[:]
  inv_s1 = 1.0 / _S1

  hs = jnp.dot(ss_ref[:], w0a, preferred_element_type=f32)
  hs += jnp.dot(sums_ref[:] * inv_s1, w0b, preferred_element_type=f32)
  hs = _l2norm(jnp.maximum(hs + b0, 0.0))

  hn = jnp.dot(sn_ref[:], w0a, preferred_element_type=f32)
  hn += jnp.dot(sumn_ref[:] * inv_s1, w0b, preferred_element_type=f32)
  hn = _l2norm(jnp.maximum(hn + b0, 0.0))

  m = jnp.mean(hn.reshape(_BS, _S2, _D), axis=1)
  w1 = w1_ref[:]
  z = jnp.dot(hs, w1[:_D], preferred_element_type=f32)
  z += jnp.dot(m, w1[_D:], preferred_element_type=f32)
  o_ref[:] = _l2norm(jnp.maximum(z + b1_ref[:], 0.0))


def _tc_mlp(self_seed, sum_seed, self_n1, sum_n1, W0, b0, W1, b1):
  grid = (_B // _BS,)
  rep = lambda i: (0, 0)
  return pl.pallas_call(
      _tc_body,
      grid=grid,
      in_specs=[
          pl.BlockSpec((_BS, _D), lambda i: (i, 0)),
          pl.BlockSpec((_BS, _D), lambda i: (i, 0)),
          pl.BlockSpec((_BS * _S2, _D), lambda i: (i, 0)),
          pl.BlockSpec((_BS * _S2, _D), lambda i: (i, 0)),
          pl.BlockSpec((2 * _D, _D), rep),
          pl.BlockSpec((1, _D), rep),
          pl.BlockSpec((2 * _D, _D), rep),
          pl.BlockSpec((1, _D), rep),
      ],
      out_specs=pl.BlockSpec((_BS, _D), lambda i: (i, 0)),
      out_shape=jax.ShapeDtypeStruct((_B, _D), jnp.float32),
  )(self_seed, sum_seed, self_n1, sum_n1, W0, b0, W1, b1)


@jax.jit
def kernel(feature, nodes, n1, nn_seed, nn_n1, W0, b0, W1, b1):
  nodes = nodes.reshape(-1).astype(jnp.int32)
  n1f = n1.reshape(-1).astype(jnp.int32)
  nnseedf = nn_seed.reshape(-1).astype(jnp.int32)
  nnn1f = nn_n1.reshape(-1).astype(jnp.int32)
  sum_n1, sum_seed, self_n1, self_seed = _sc_gather(
      feature, nnn1f, nnseedf, n1f, nodes)
  return _tc_mlp(self_seed, sum_seed, self_n1, sum_n1,
                 W0, b0.reshape(1, _D), W1, b1.reshape(1, _D))


# speedup vs baseline: 9.8900x; 1.0304x over previous
import functools

import jax
import jax.numpy as jnp
from jax import lax
from jax.experimental import pallas as pl
from jax.experimental.pallas import tpu as pltpu
from jax.experimental.pallas import tpu_sc as plsc

_N, _D = 100000, 128
_B, _S1, _S2 = 1024, 25, 10

_NC, _NS = 2, 16
_NW = _NC * _NS

_PAIRS = _B * _S2 // _NW
_SEEDS = _B // _NW
_CH = 8
_ROWS = _CH * _S1
_NBUF = 4
_NCHUNK_N1 = _PAIRS // _CH
_NCHUNK_SEED = _SEEDS // _CH
_VREGS = _D // 16
_SELF_CH = _PAIRS // _NBUF


def _sc_gather_body(feat, nnn1, nnseed, n1f, nodes,
                    sum_n1, sum_seed, self_n1, self_seed,
                    idx_v, buf0, buf1, buf2, buf3, stage,
                    sem0, sem1, sem2, sem3, osem):
  wid = lax.axis_index("c") * _NS + lax.axis_index("s")
  bufs = (buf0, buf1, buf2, buf3)
  sems = (sem0, sem1, sem2, sem3)

  def gather_pieces(rows):
    pieces, o = [], 0
    while o < rows:
      ln = min(128, rows - o)
      pieces.append((o, ln))
      o += ln
    return pieces

  def gather_start(idx_off, buf, sem, rows):
    for o, ln in gather_pieces(rows):
      pltpu.async_copy(feat.at[idx_v.at[pl.ds(idx_off + o, ln)]],
                       buf.at[pl.ds(o, ln)], sem)

  def gather_wait(idx_off, buf, sem, rows):
    for o, ln in gather_pieces(rows):
      pltpu.make_async_copy(feat.at[idx_v.at[pl.ds(idx_off + o, ln)]],
                            buf.at[pl.ds(o, ln)], sem).wait()

  def acc_group(buf, stage_base):
    def pair_body(p, carry):
      s = [buf[p * _S1, pl.ds(v * 16, 16)] for v in range(_VREGS)]
      for j in range(1, _S1):
        for v in range(_VREGS):
          s[v] = s[v] + buf[p * _S1 + j, pl.ds(v * 16, 16)]
      for v in range(_VREGS):
        stage[stage_base + p, pl.ds(v * 16, 16)] = s[v]
      return carry
    lax.fori_loop(0, _CH, pair_body, 0)

  def stage_write(b, g, out_hbm, out_base):
    return pltpu.make_async_copy(
        stage.at[pl.ds(b * _CH, _CH)],
        out_hbm.at[pl.ds(out_base + g * _CH, _CH)], osem)

  def sum_phase(idx_hbm, idx_count, nchunks, out_hbm, out_base):
    base = wid * idx_count
    pltpu.sync_copy(idx_hbm.at[pl.ds(base, idx_count)],
                    idx_v.at[pl.ds(0, idx_count)])
    for b in range(_NBUF):
      gather_start(b * _ROWS, bufs[b], sems[b], _ROWS)

    def loop_body(k, carry):
      g0 = _NBUF * k
      for b in range(_NBUF):
        g = g0 + b
        gather_wait(g * _ROWS, bufs[b], sems[b], _ROWS)

        @pl.when(g >= _NBUF)
        def _():
          stage_write(b, g - _NBUF, out_hbm, out_base).wait()

        acc_group(bufs[b], b * _CH)
        stage_write(b, g, out_hbm, out_base).start()

        @pl.when(g + _NBUF < nchunks)
        def _():
          gather_start((g + _NBUF) * _ROWS, bufs[b], sems[b], _ROWS)
      return carry

    lax.fori_loop(0, nchunks // _NBUF, loop_body, 0)
    for b in range(_NBUF):
      stage_write(b, nchunks - _NBUF + b, out_hbm, out_base).wait()

  sum_phase(nnn1, _PAIRS * _S1, _NCHUNK_N1, sum_n1, wid * _PAIRS)

  sum_phase(nnseed, _SEEDS * _S1, _NCHUNK_SEED, sum_seed, wid * _SEEDS)

  pltpu.sync_copy(n1f.at[pl.ds(wid * _PAIRS, _PAIRS)],
                  idx_v.at[pl.ds(0, _PAIRS)])
  for b in range(_NBUF):
    gather_start(b * _SELF_CH, bufs[b], sems[b], _SELF_CH)
  for b in range(_NBUF):
    gather_wait(b * _SELF_CH, bufs[b], sems[b], _SELF_CH)
    pltpu.async_copy(
        bufs[b].at[pl.ds(0, _SELF_CH)],
        self_n1.at[pl.ds(wid * _PAIRS + b * _SELF_CH, _SELF_CH)], osem)

  pltpu.sync_copy(nodes.at[pl.ds(wid * _SEEDS, _SEEDS)],
                  idx_v.at[pl.ds(0, _SEEDS)])
  gather_start(0, stage, sem0, _SEEDS)
  gather_wait(0, stage, sem0, _SEEDS)
  pltpu.async_copy(stage, self_seed.at[pl.ds(wid * _SEEDS, _SEEDS)], osem)

  for b in range(_NBUF):
    pltpu.make_async_copy(
        bufs[b].at[pl.ds(0, _SELF_CH)],
        self_n1.at[pl.ds(wid * _PAIRS + b * _SELF_CH, _SELF_CH)], osem).wait()
  pltpu.make_async_copy(stage, self_seed.at[pl.ds(wid * _SEEDS, _SEEDS)],
                        osem).wait()


def _sc_gather(feature, nnn1f, nnseedf, n1f, nodes):
  mesh = plsc.VectorSubcoreMesh(core_axis_name="c", subcore_axis_name="s",
                                num_cores=_NC, num_subcores=_NS)
  f32 = jnp.float32
  return pl.kernel(
      _sc_gather_body,
      out_type=(
          jax.ShapeDtypeStruct((_B * _S2, _D), f32),
          jax.ShapeDtypeStruct((_B, _D), f32),
          jax.ShapeDtypeStruct((_B * _S2, _D), f32),
          jax.ShapeDtypeStruct((_B, _D), f32),
      ),
      mesh=mesh,
      scratch_types=[
          pltpu.VMEM((_PAIRS * _S1,), jnp.int32),
          pltpu.VMEM((_ROWS, _D), f32),
          pltpu.VMEM((_ROWS, _D), f32),
          pltpu.VMEM((_ROWS, _D), f32),
          pltpu.VMEM((_ROWS, _D), f32),
          pltpu.VMEM((_NBUF * _CH, _D), f32),
          pltpu.SemaphoreType.DMA,
          pltpu.SemaphoreType.DMA,
          pltpu.SemaphoreType.DMA,
          pltpu.SemaphoreType.DMA,
          pltpu.SemaphoreType.DMA,
      ],
  )(feature, nnn1f, nnseedf, n1f, nodes)


_BS = 128


def _l2norm(h):
  ss = jnp.sum(h * h, axis=-1, keepdims=True)
  return h / jnp.maximum(jnp.sqrt(ss), 1e-12)


def _tc_body(ss_ref, sums_ref, sn_ref, sumn_ref, w0_ref, b0_ref, w1_ref,
             b1_ref, o_ref):
  f32 = jnp.float32
  w0 = w0_ref[:]
  w0a, w0b = w0[:_D], w0[_D:]
  b0 = b0_ref[:]
  inv_s1 = 1.0 / _S1

  hs = jnp.dot(ss_ref[:], w0a, preferred_element_type=f32)
  hs += jnp.dot(sums_ref[:] * inv_s1, w0b, preferred_element_type=f32)
  hs = _l2norm(jnp.maximum(hs + b0, 0.0))

  hn = jnp.dot(sn_ref[:], w0a, preferred_element_type=f32)
  hn += jnp.dot(sumn_ref[:] * inv_s1, w0b, preferred_element_type=f32)
  hn = _l2norm(jnp.maximum(hn + b0, 0.0))

  m = jnp.mean(hn.reshape(_BS, _S2, _D), axis=1)
  w1 = w1_ref[:]
  z = jnp.dot(hs, w1[:_D], preferred_element_type=f32)
  z += jnp.dot(m, w1[_D:], preferred_element_type=f32)
  o_ref[:] = _l2norm(jnp.maximum(z + b1_ref[:], 0.0))


def _tc_mlp(self_seed, sum_seed, self_n1, sum_n1, W0, b0, W1, b1):
  grid = (_B // _BS,)
  rep = lambda i: (0, 0)
  return pl.pallas_call(
      _tc_body,
      grid=grid,
      in_specs=[
          pl.BlockSpec((_BS, _D), lambda i: (i, 0)),
          pl.BlockSpec((_BS, _D), lambda i: (i, 0)),
          pl.BlockSpec((_BS * _S2, _D), lambda i: (i, 0)),
          pl.BlockSpec((_BS * _S2, _D), lambda i: (i, 0)),
          pl.BlockSpec((2 * _D, _D), rep),
          pl.BlockSpec((1, _D), rep),
          pl.BlockSpec((2 * _D, _D), rep),
          pl.BlockSpec((1, _D), rep),
      ],
      out_specs=pl.BlockSpec((_BS, _D), lambda i: (i, 0)),
      out_shape=jax.ShapeDtypeStruct((_B, _D), jnp.float32),
  )(self_seed, sum_seed, self_n1, sum_n1, W0, b0, W1, b1)


@jax.jit
def kernel(feature, nodes, n1, nn_seed, nn_n1, W0, b0, W1, b1):
  nodes = nodes.reshape(-1).astype(jnp.int32)
  n1f = n1.reshape(-1).astype(jnp.int32)
  nnseedf = nn_seed.reshape(-1).astype(jnp.int32)
  nnn1f = nn_n1.reshape(-1).astype(jnp.int32)
  sum_n1, sum_seed, self_n1, self_seed = _sc_gather(
      feature, nnn1f, nnseedf, n1f, nodes)
  return _tc_mlp(self_seed, sum_seed, self_n1, sum_n1,
                 W0, b0.reshape(1, _D), W1, b1.reshape(1, _D))


# speedup vs baseline: 10.5438x; 1.0661x over previous
import functools

import jax
import jax.numpy as jnp
from jax import lax
from jax.experimental import pallas as pl
from jax.experimental.pallas import tpu as pltpu
from jax.experimental.pallas import tpu_sc as plsc

_N, _D = 100000, 128
_B, _S1, _S2 = 1024, 25, 10

_NC, _NS = 2, 16
_NW = _NC * _NS

_PAIRS = _B * _S2 // _NW
_SEEDS = _B // _NW
_CH = 8
_ROWS = _CH * _S1
_NBUF = 4
_NCH_N1 = _PAIRS // _CH
_NCH_SUM = (_PAIRS + _SEEDS) // _CH
_VREGS = _D // 16

_OFF_SELF = (_PAIRS + _SEEDS) * _S1
_OFF_NODE = _OFF_SELF + _PAIRS
_WBLK = _OFF_NODE + _SEEDS
_SELF_H = 176


def _sc_gather_body(feat, idx_all,
                    sum_n1, sum_seed, self_n1, self_seed,
                    idx_v, buf0, buf1, buf2, buf3, stage,
                    sem0, sem1, sem2, sem3, osem):
  wid = lax.axis_index("c") * _NS + lax.axis_index("s")
  bufs = (buf0, buf1, buf2, buf3)
  sems = (sem0, sem1, sem2, sem3)

  def gather_pieces(rows):
    pieces, o = [], 0
    while o < rows:
      ln = min(128, rows - o)
      pieces.append((o, ln))
      o += ln
    return pieces

  def gather_start(idx_off, buf, sem, rows):
    for o, ln in gather_pieces(rows):
      pltpu.async_copy(feat.at[idx_v.at[pl.ds(idx_off + o, ln)]],
                       buf.at[pl.ds(o, ln)], sem)

  def gather_wait(idx_off, buf, sem, rows):
    for o, ln in gather_pieces(rows):
      pltpu.make_async_copy(feat.at[idx_v.at[pl.ds(idx_off + o, ln)]],
                            buf.at[pl.ds(o, ln)], sem).wait()

  def acc_group(buf, stage_base):
    def pair_body(p, carry):
      s = [buf[p * _S1, pl.ds(v * 16, 16)] for v in range(_VREGS)]
      for j in range(1, _S1):
        for v in range(_VREGS):
          s[v] = s[v] + buf[p * _S1 + j, pl.ds(v * 16, 16)]
      for v in range(_VREGS):
        stage[stage_base + p, pl.ds(v * 16, 16)] = s[v]
      return carry
    lax.fori_loop(0, _CH, pair_body, 0)

  def n1_write(b, g):
    return pltpu.make_async_copy(
        stage.at[pl.ds(b * _CH, _CH)],
        sum_n1.at[pl.ds(wid * _PAIRS + g * _CH, _CH)], osem)

  def seed_write(b, g):
    return pltpu.make_async_copy(
        stage.at[pl.ds(b * _CH, _CH)],
        sum_seed.at[pl.ds(wid * _SEEDS + (g - _NCH_N1) * _CH, _CH)], osem)

  pltpu.sync_copy(idx_all.at[pl.ds(wid * _WBLK, _WBLK)], idx_v)
  for b in range(_NBUF):
    gather_start(b * _ROWS, bufs[b], sems[b], _ROWS)

  def loop_body(k, carry):
    g0 = _NBUF * k
    for b in range(_NBUF):
      g = g0 + b
      gather_wait(g * _ROWS, bufs[b], sems[b], _ROWS)

      @pl.when(g >= _NBUF)
      def _():
        n1_write(b, g - _NBUF).wait()

      acc_group(bufs[b], b * _CH)

      @pl.when(g < _NCH_N1)
      def _():
        n1_write(b, g).start()

      @pl.when(g >= _NCH_N1)
      def _():
        seed_write(b, g).start()

      @pl.when(g + _NBUF < _NCH_SUM)
      def _():
        gather_start((g + _NBUF) * _ROWS, bufs[b], sems[b], _ROWS)
    return carry

  lax.fori_loop(0, _NCH_SUM // _NBUF, loop_body, 0)
  for b in range(_NBUF):
    seed_write(b, _NCH_N1 + b).wait()

  gather_start(_OFF_SELF, buf0, sem0, _SELF_H)
  gather_start(_OFF_SELF + _SELF_H, buf1, sem1, _SELF_H)
  gather_wait(_OFF_SELF, buf0, sem0, _SELF_H)
  pltpu.async_copy(buf0.at[pl.ds(0, _SELF_H)],
                   self_n1.at[pl.ds(wid * _PAIRS, _SELF_H)], osem)
  gather_wait(_OFF_SELF + _SELF_H, buf1, sem1, _SELF_H)
  n1_rest = _PAIRS - _SELF_H
  pltpu.async_copy(buf1.at[pl.ds(0, n1_rest)],
                   self_n1.at[pl.ds(wid * _PAIRS + _SELF_H, n1_rest)], osem)
  pltpu.async_copy(buf1.at[pl.ds(n1_rest, _SEEDS)],
                   self_seed.at[pl.ds(wid * _SEEDS, _SEEDS)], osem)

  pltpu.make_async_copy(buf0.at[pl.ds(0, _SELF_H)],
                        self_n1.at[pl.ds(wid * _PAIRS, _SELF_H)], osem).wait()
  pltpu.make_async_copy(buf1.at[pl.ds(0, n1_rest)],
                        self_n1.at[pl.ds(wid * _PAIRS + _SELF_H, n1_rest)],
                        osem).wait()
  pltpu.make_async_copy(buf1.at[pl.ds(n1_rest, _SEEDS)],
                        self_seed.at[pl.ds(wid * _SEEDS, _SEEDS)], osem).wait()


def _sc_gather(feature, idx_all):
  mesh = plsc.VectorSubcoreMesh(core_axis_name="c", subcore_axis_name="s",
                                num_cores=_NC, num_subcores=_NS)
  f32 = jnp.float32
  return pl.kernel(
      _sc_gather_body,
      out_type=(
          jax.ShapeDtypeStruct((_B * _S2, _D), f32),
          jax.ShapeDtypeStruct((_B, _D), f32),
          jax.ShapeDtypeStruct((_B * _S2, _D), f32),
          jax.ShapeDtypeStruct((_B, _D), f32),
      ),
      mesh=mesh,
      scratch_types=[
          pltpu.VMEM((_WBLK,), jnp.int32),
          pltpu.VMEM((_ROWS, _D), f32),
          pltpu.VMEM((_ROWS, _D), f32),
          pltpu.VMEM((_ROWS, _D), f32),
          pltpu.VMEM((_ROWS, _D), f32),
          pltpu.VMEM((_NBUF * _CH, _D), f32),
          pltpu.SemaphoreType.DMA,
          pltpu.SemaphoreType.DMA,
          pltpu.SemaphoreType.DMA,
          pltpu.SemaphoreType.DMA,
          pltpu.SemaphoreType.DMA,
      ],
  )(feature, idx_all)


_BS = 128


def _l2norm(h):
  ss = jnp.sum(h * h, axis=-1, keepdims=True)
  return h / jnp.maximum(jnp.sqrt(ss), 1e-12)


def _tc_body(ss_ref, sums_ref, sn_ref, sumn_ref, w0_ref, b0_ref, w1_ref,
             b1_ref, o_ref):
  f32 = jnp.float32
  w0 = w0_ref[:]
  w0a, w0b = w0[:_D], w0[_D:]
  b0 = b0_ref[:]
  inv_s1 = 1.0 / _S1

  hs = jnp.dot(ss_ref[:], w0a, preferred_element_type=f32)
  hs += jnp.dot(sums_ref[:] * inv_s1, w0b, preferred_element_type=f32)
  hs = _l2norm(jnp.maximum(hs + b0, 0.0))

  hn = jnp.dot(sn_ref[:], w0a, preferred_element_type=f32)
  hn += jnp.dot(sumn_ref[:] * inv_s1, w0b, preferred_element_type=f32)
  hn = _l2norm(jnp.maximum(hn + b0, 0.0))

  m = jnp.mean(hn.reshape(_BS, _S2, _D), axis=1)
  w1 = w1_ref[:]
  z = jnp.dot(hs, w1[:_D], preferred_element_type=f32)
  z += jnp.dot(m, w1[_D:], preferred_element_type=f32)
  o_ref[:] = _l2norm(jnp.maximum(z + b1_ref[:], 0.0))


def _tc_mlp(self_seed, sum_seed, self_n1, sum_n1, W0, b0, W1, b1):
  grid = (_B // _BS,)
  rep = lambda i: (0, 0)
  return pl.pallas_call(
      _tc_body,
      grid=grid,
      in_specs=[
          pl.BlockSpec((_BS, _D), lambda i: (i, 0)),
          pl.BlockSpec((_BS, _D), lambda i: (i, 0)),
          pl.BlockSpec((_BS * _S2, _D), lambda i: (i, 0)),
          pl.BlockSpec((_BS * _S2, _D), lambda i: (i, 0)),
          pl.BlockSpec((2 * _D, _D), rep),
          pl.BlockSpec((1, _D), rep),
          pl.BlockSpec((2 * _D, _D), rep),
          pl.BlockSpec((1, _D), rep),
      ],
      out_specs=pl.BlockSpec((_BS, _D), lambda i: (i, 0)),
      out_shape=jax.ShapeDtypeStruct((_B, _D), jnp.float32),
  )(self_seed, sum_seed, self_n1, sum_n1, W0, b0, W1, b1)


@jax.jit
def kernel(feature, nodes, n1, nn_seed, nn_n1, W0, b0, W1, b1):
  i32 = jnp.int32
  idx_all = jnp.concatenate([
      nn_n1.astype(i32).reshape(_NW, -1),
      nn_seed.astype(i32).reshape(_NW, -1),
      n1.astype(i32).reshape(_NW, -1),
      nodes.astype(i32).reshape(_NW, -1),
  ], axis=1).reshape(-1)
  sum_n1, sum_seed, self_n1, self_seed = _sc_gather(feature, idx_all)
  return _tc_mlp(self_seed, sum_seed, self_n1, sum_n1,
                 W0, b0.reshape(1, _D), W1, b1.reshape(1, _D))


# speedup vs baseline: 10.6831x; 1.0132x over previous
import functools

import jax
import jax.numpy as jnp
from jax import lax
from jax.experimental import pallas as pl
from jax.experimental.pallas import tpu as pltpu
from jax.experimental.pallas import tpu_sc as plsc

_N, _D = 100000, 128
_B, _S1, _S2 = 1024, 25, 10

_NC, _NS = 2, 16
_NW = _NC * _NS

_PAIRS = _B * _S2 // _NW
_SEEDS = _B // _NW
_CH = 8
_ROWS = _CH * _S1
_NBUF = 4
_NCH_N1 = _PAIRS // _CH
_NCH_SEED = _SEEDS // _CH
_VREGS = _D // 16

_A_SELF = _SEEDS * _S1
_A_NODE = _A_SELF + _PAIRS
_A_BLK = _A_NODE + _SEEDS
_SELF_H = 176

_MESH = plsc.VectorSubcoreMesh(core_axis_name="c", subcore_axis_name="s",
                               num_cores=_NC, num_subcores=_NS)


def _worker_id():
  return lax.axis_index("c") * _NS + lax.axis_index("s")


def _gather_pieces(rows):
  pieces, o = [], 0
  while o < rows:
    ln = min(128, rows - o)
    pieces.append((o, ln))
    o += ln
  return pieces


def _gather_start(feat, idx_v, idx_off, buf, sem, rows):
  for o, ln in _gather_pieces(rows):
    pltpu.async_copy(feat.at[idx_v.at[pl.ds(idx_off + o, ln)]],
                     buf.at[pl.ds(o, ln)], sem)


def _gather_wait(feat, idx_v, idx_off, buf, sem, rows):
  for o, ln in _gather_pieces(rows):
    pltpu.make_async_copy(feat.at[idx_v.at[pl.ds(idx_off + o, ln)]],
                          buf.at[pl.ds(o, ln)], sem).wait()


def _acc_group(buf, stage, stage_base):
  def pair_body(p, carry):
    s = [buf[p * _S1, pl.ds(v * 16, 16)] for v in range(_VREGS)]
    for j in range(1, _S1):
      for v in range(_VREGS):
        s[v] = s[v] + buf[p * _S1 + j, pl.ds(v * 16, 16)]
    for v in range(_VREGS):
      stage[stage_base + p, pl.ds(v * 16, 16)] = s[v]
    return carry
  lax.fori_loop(0, _CH, pair_body, 0)


def _sc_small_body(feat, idx_all, sum_seed, self_n1, self_seed,
                   idx_v, buf0, buf1, buf2, buf3, stage,
                   sem0, sem1, sem2, sem3, osem):
  wid = _worker_id()
  bufs = (buf0, buf1, buf2, buf3)
  sems = (sem0, sem1, sem2, sem3)

  pltpu.sync_copy(idx_all.at[pl.ds(wid * _A_BLK, _A_BLK)], idx_v)
  for b in range(_NCH_SEED):
    _gather_start(feat, idx_v, b * _ROWS, bufs[b], sems[b], _ROWS)
  self_writes = []
  for h in range(2):
    buf, sem = bufs[h], sems[h]
    _gather_wait(feat, idx_v, h * _ROWS, buf, sem, _ROWS)
    _acc_group(buf, stage, h * _CH)
    w = pltpu.make_async_copy(
        stage.at[pl.ds(h * _CH, _CH)],
        sum_seed.at[pl.ds(wid * _SEEDS + h * _CH, _CH)], osem)
    w.start()
    self_writes.append(w)
    _gather_start(feat, idx_v, _A_SELF + h * _SELF_H, buf, sem, _SELF_H)
  for h in (2, 3):
    _gather_wait(feat, idx_v, h * _ROWS, bufs[h], sems[h], _ROWS)
    _acc_group(bufs[h], stage, h * _CH)
    w = pltpu.make_async_copy(
        stage.at[pl.ds(h * _CH, _CH)],
        sum_seed.at[pl.ds(wid * _SEEDS + h * _CH, _CH)], osem)
    w.start()
    self_writes.append(w)

  _gather_wait(feat, idx_v, _A_SELF, buf0, sem0, _SELF_H)
  w = pltpu.make_async_copy(buf0.at[pl.ds(0, _SELF_H)],
                            self_n1.at[pl.ds(wid * _PAIRS, _SELF_H)], osem)
  w.start()
  self_writes.append(w)
  _gather_wait(feat, idx_v, _A_SELF + _SELF_H, buf1, sem1, _SELF_H)
  n1_rest = _PAIRS - _SELF_H
  w = pltpu.make_async_copy(
      buf1.at[pl.ds(0, n1_rest)],
      self_n1.at[pl.ds(wid * _PAIRS + _SELF_H, n1_rest)], osem)
  w.start()
  self_writes.append(w)
  w = pltpu.make_async_copy(buf1.at[pl.ds(n1_rest, _SEEDS)],
                            self_seed.at[pl.ds(wid * _SEEDS, _SEEDS)], osem)
  w.start()
  self_writes.append(w)
  for w in self_writes:
    w.wait()


def _sc_n1_body(feat, nnn1, sum_n1,
                idx_v, buf0, buf1, buf2, buf3, stage,
                sem0, sem1, sem2, sem3, osem):
  wid = _worker_id()
  bufs = (buf0, buf1, buf2, buf3)
  sems = (sem0, sem1, sem2, sem3)

  def n1_write(b, g):
    return pltpu.make_async_copy(
        stage.at[pl.ds(b * _CH, _CH)],
        sum_n1.at[pl.ds(wid * _PAIRS + g * _CH, _CH)], osem)

  pltpu.sync_copy(nnn1.at[pl.ds(wid * _PAIRS * _S1, _PAIRS * _S1)], idx_v)
  for b in range(_NBUF):
    _gather_start(feat, idx_v, b * _ROWS, bufs[b], sems[b], _ROWS)

  def loop_body(k, carry):
    g0 = _NBUF * k
    for b in range(_NBUF):
      g = g0 + b
      _gather_wait(feat, idx_v, g * _ROWS, bufs[b], sems[b], _ROWS)

      @pl.when(g >= _NBUF)
      def _():
        n1_write(b, g - _NBUF).wait()

      _acc_group(bufs[b], stage, b * _CH)
      n1_write(b, g).start()

      @pl.when(g + _NBUF < _NCH_N1)
      def _():
        _gather_start(feat, idx_v, (g + _NBUF) * _ROWS, bufs[b], sems[b],
                      _ROWS)
    return carry

  lax.fori_loop(0, _NCH_N1 // _NBUF, loop_body, 0)
  for b in range(_NBUF):
    n1_write(b, _NCH_N1 - _NBUF + b).wait()


def _sc_scratch(idx_len):
  f32 = jnp.float32
  return [
      pltpu.VMEM((idx_len,), jnp.int32),
      pltpu.VMEM((_ROWS, _D), f32),
      pltpu.VMEM((_ROWS, _D), f32),
      pltpu.VMEM((_ROWS, _D), f32),
      pltpu.VMEM((_ROWS, _D), f32),
      pltpu.VMEM((_NBUF * _CH, _D), f32),
      pltpu.SemaphoreType.DMA,
      pltpu.SemaphoreType.DMA,
      pltpu.SemaphoreType.DMA,
      pltpu.SemaphoreType.DMA,
      pltpu.SemaphoreType.DMA,
  ]


def _sc_small(feature, idx_all):
  f32 = jnp.float32
  return pl.kernel(
      _sc_small_body,
      out_type=(
          jax.ShapeDtypeStruct((_B, _D), f32),
          jax.ShapeDtypeStruct((_B * _S2, _D), f32),
          jax.ShapeDtypeStruct((_B, _D), f32),
      ),
      mesh=_MESH,
      scratch_types=_sc_scratch(_A_BLK),
  )(feature, idx_all)


def _sc_n1(feature, nnn1f):
  return pl.kernel(
      _sc_n1_body,
      out_type=jax.ShapeDtypeStruct((_B * _S2, _D), jnp.float32),
      mesh=_MESH,
      scratch_types=_sc_scratch(_PAIRS * _S1),
  )(feature, nnn1f)


_BS = 128


def _l2norm(h):
  ss = jnp.sum(h * h, axis=-1, keepdims=True)
  return h / jnp.maximum(jnp.sqrt(ss), 1e-12)


def _tc_body(ss_ref, sums_ref, sn_ref, sumn_ref, w0_ref, b0_ref, w1_ref,
             b1_ref, o_ref):
  f32 = jnp.float32
  w0 = w0_ref[:]
  w0a, w0b = w0[:_D], w0[_D:]
  b0 = b0_ref[:]
  inv_s1 = 1.0 / _S1

  hs = jnp.dot(ss_ref[:], w0a, preferred_element_type=f32)
  hs += jnp.dot(sums_ref[:] * inv_s1, w0b, preferred_element_type=f32)
  hs = _l2norm(jnp.maximum(hs + b0, 0.0))

  hn = jnp.dot(sn_ref[:], w0a, preferred_element_type=f32)
  hn += jnp.dot(sumn_ref[:] * inv_s1, w0b, preferred_element_type=f32)
  hn = _l2norm(jnp.maximum(hn + b0, 0.0))

  m = jnp.mean(hn.reshape(_BS, _S2, _D), axis=1)
  w1 = w1_ref[:]
  z = jnp.dot(hs, w1[:_D], preferred_element_type=f32)
  z += jnp.dot(m, w1[_D:], preferred_element_type=f32)
  o_ref[:] = _l2norm(jnp.maximum(z + b1_ref[:], 0.0))


def _tc_mlp(self_seed, sum_seed, self_n1, sum_n1, W0, b0, W1, b1):
  grid = (_B // _BS,)
  rep = lambda i: (0, 0)
  return pl.pallas_call(
      _tc_body,
      grid=grid,
      in_specs=[
          pl.BlockSpec((_BS, _D), lambda i: (i, 0)),
          pl.BlockSpec((_BS, _D), lambda i: (i, 0)),
          pl.BlockSpec((_BS * _S2, _D), lambda i: (i, 0)),
          pl.BlockSpec((_BS * _S2, _D), lambda i: (i, 0)),
          pl.BlockSpec((2 * _D, _D), rep),
          pl.BlockSpec((1, _D), rep),
          pl.BlockSpec((2 * _D, _D), rep),
          pl.BlockSpec((1, _D), rep),
      ],
      out_specs=pl.BlockSpec((_BS, _D), lambda i: (i, 0)),
      out_shape=jax.ShapeDtypeStruct((_B, _D), jnp.float32),
  )(self_seed, sum_seed, self_n1, sum_n1, W0, b0, W1, b1)


@jax.jit
def kernel(feature, nodes, n1, nn_seed, nn_n1, W0, b0, W1, b1):
  i32 = jnp.int32
  idx_a = jnp.concatenate([
      nn_seed.astype(i32).reshape(_NW, -1),
      n1.astype(i32).reshape(_NW, -1),
      nodes.astype(i32).reshape(_NW, -1),
  ], axis=1).reshape(-1)
  nnn1f = nn_n1.astype(i32).reshape(-1)
  sum_seed, self_n1, self_seed = _sc_small(feature, idx_a)
  sum_n1 = _sc_n1(feature, nnn1f)
  return _tc_mlp(self_seed, sum_seed, self_n1, sum_n1,
                 W0, b0.reshape(1, _D), W1, b1.reshape(1, _D))
